# Initial kernel scaffold; baseline (speedup 1.0000x reference)
import functools

import jax
import jax.numpy as jnp
from jax import lax
from jax.experimental import pallas as pl
from jax.experimental.pallas import tpu as pltpu
from jax.experimental.pallas import tpu_sc as plsc

N_ROWS = 262144
N_COLS = 128
NUM_CORES = 2
NUM_SUBCORES = 16
NUM_WORKERS = NUM_CORES * NUM_SUBCORES
ROWS_PER_WORKER = N_ROWS // NUM_WORKERS
LANES = 16
ITERS = ROWS_PER_WORKER // LANES


def _sc_kernel(ram_hbm, out_hbm, xy_v, clk_v, ind_v):
    cid = lax.axis_index("c")
    sid = lax.axis_index("s")
    wid = sid * NUM_CORES + cid
    row0 = wid * ROWS_PER_WORKER

    pltpu.sync_copy(ram_hbm.at[pl.ds(row0, ROWS_PER_WORKER), pl.ds(32, 4)], xy_v)
    pltpu.sync_copy(ram_hbm.at[pl.ds(row0, ROWS_PER_WORKER), pl.ds(0, 4)], clk_v)

    iot = lax.iota(jnp.int32, LANES)
    zero = jnp.zeros((LANES,), jnp.int32)
    one = jnp.ones((LANES,), jnp.int32)

    def body(i, acc):
        ridx = i * LANES + iot
        mi_x = plsc.load_gather(xy_v, [ridx, zero])
        su_x = plsc.load_gather(xy_v, [ridx, zero + 1])
        mi_y = plsc.load_gather(xy_v, [ridx, zero + 2])
        su_y = plsc.load_gather(xy_v, [ridx, zero + 3])
        clk = plsc.load_gather(clk_v, [ridx, zero])
        dist_x = jnp.abs(su_x - mi_x)
        dist_y = jnp.abs(su_y - mi_y)
        go_down = su_y > mi_y + 1
        go_right = su_x > mi_x
        punch = (clk % 12) < 4
        d2 = dist_y <= 2
        act = jnp.where(go_down, 5, 2)
        act = jnp.where(d2 & (dist_x > 26), jnp.where(go_right, 3, 4), act)
        act = jnp.where(d2 & (dist_x < 23), jnp.where(go_right, 4, 3), act)
        act = jnp.where(
            d2 & (dist_x >= 23) & (dist_x <= 26), jnp.where(punch, 1, 0), act
        )
        return acc | (one << act)

    acc = lax.fori_loop(0, ITERS, body, jnp.zeros((LANES,), jnp.int32))

    ind = zero
    for k in range(6):
        seen_k = jnp.max((acc >> k) & 1)
        ind = jnp.where(iot == k, seen_k, ind)
    ind_v[...] = ind.astype(jnp.float32)
    pltpu.sync_copy(ind_v, out_hbm.at[wid])


@jax.jit
def _run(ram):
    mesh = plsc.VectorSubcoreMesh(core_axis_name="c", subcore_axis_name="s")
    k = functools.partial(
        pl.kernel,
        mesh=mesh,
        out_type=jax.ShapeDtypeStruct((NUM_WORKERS, LANES), jnp.float32),
        scratch_types=[
            pltpu.VMEM((ROWS_PER_WORKER, 4), jnp.int32),
            pltpu.VMEM((ROWS_PER_WORKER, 4), jnp.int32),
            pltpu.VMEM((LANES,), jnp.float32),
        ],
    )(_sc_kernel)
    partial_ind = k(ram)
    merged = jnp.max(partial_ind, axis=0)
    logits = jnp.zeros((1, 18), dtype=jnp.float32)
    return lax.dynamic_update_slice(logits, merged.reshape(1, LANES), (0, 0))


def kernel(ram):
    return _run(ram)


# speedup vs baseline: 14.4682x; 14.4682x over previous
import functools

import jax
import jax.numpy as jnp
from jax import lax
from jax.experimental import pallas as pl
from jax.experimental.pallas import tpu as pltpu
from jax.experimental.pallas import tpu_sc as plsc

N_ROWS = 262144
N_COLS = 128
NUM_CORES = 2
NUM_SUBCORES = 16
NUM_WORKERS = NUM_CORES * NUM_SUBCORES
RPW = N_ROWS // NUM_WORKERS
L = 16


def _sc_body(view_hbm, out_hbm, imx, isx, imy, isy, iclk, bmx, bsx, bmy, bsy, bclk, acc_v, sem):
    cid = lax.axis_index("c")
    sid = lax.axis_index("s")
    wid = sid * NUM_CORES + cid
    row0 = wid * RPW
    iot = lax.iota(jnp.int32, L)

    def fill(j, carry):
        base = N_COLS * (row0 + j * L + iot)
        imx[pl.ds(j * L, L)] = base + 32
        isx[pl.ds(j * L, L)] = base + 33
        imy[pl.ds(j * L, L)] = base + 34
        isy[pl.ds(j * L, L)] = base + 35
        iclk[pl.ds(j * L, L)] = base
        return carry

    lax.fori_loop(0, RPW // L, fill, 0)

    pltpu.async_copy(view_hbm.at[imx], bmx, sem)
    pltpu.async_copy(view_hbm.at[isx], bsx, sem)
    pltpu.async_copy(view_hbm.at[imy], bmy, sem)
    pltpu.async_copy(view_hbm.at[isy], bsy, sem)
    pltpu.async_copy(view_hbm.at[iclk], bclk, sem)
    pltpu.make_async_copy(view_hbm.at[imx], bmx, sem).wait()
    pltpu.make_async_copy(view_hbm.at[isx], bsx, sem).wait()
    pltpu.make_async_copy(view_hbm.at[imy], bmy, sem).wait()
    pltpu.make_async_copy(view_hbm.at[isy], bsy, sem).wait()
    pltpu.make_async_copy(view_hbm.at[iclk], bclk, sem).wait()

    one = jnp.ones((L,), jnp.int32)

    def body16(i, acc):
        s = pl.ds(i * L, L)
        mi_x = bmx[s]
        su_x = bsx[s]
        mi_y = bmy[s]
        su_y = bsy[s]
        clk = bclk[s]
        dist_x = jnp.abs(su_x - mi_x)
        dist_y = jnp.abs(su_y - mi_y)
        go_down = su_y > mi_y + 1
        go_right = su_x > mi_x
        punch = (clk % 12) < 4
        d2 = dist_y <= 2
        act = jnp.where(go_down, 5, 2)
        act = jnp.where(d2 & (dist_x > 26), jnp.where(go_right, 3, 4), act)
        act = jnp.where(d2 & (dist_x < 23), jnp.where(go_right, 4, 3), act)
        act = jnp.where(
            d2 & (dist_x >= 23) & (dist_x <= 26), jnp.where(punch, 1, 0), act
        )
        return acc | (one << act)

    acc = lax.fori_loop(0, RPW // L, body16, jnp.zeros((L,), jnp.int32))
    acc_v[...] = acc
    pltpu.sync_copy(acc_v, out_hbm.at[wid])


@jax.jit
def _run(ram):
    view = ram.reshape(N_ROWS * N_COLS)
    mesh = plsc.VectorSubcoreMesh(core_axis_name="c", subcore_axis_name="s")
    k = functools.partial(
        pl.kernel,
        mesh=mesh,
        out_type=jax.ShapeDtypeStruct((NUM_WORKERS, L), jnp.int32),
        scratch_types=[
            pltpu.VMEM((RPW,), jnp.int32),
            pltpu.VMEM((RPW,), jnp.int32),
            pltpu.VMEM((RPW,), jnp.int32),
            pltpu.VMEM((RPW,), jnp.int32),
            pltpu.VMEM((RPW,), jnp.int32),
            pltpu.VMEM((RPW,), jnp.int32),
            pltpu.VMEM((RPW,), jnp.int32),
            pltpu.VMEM((RPW,), jnp.int32),
            pltpu.VMEM((RPW,), jnp.int32),
            pltpu.VMEM((RPW,), jnp.int32),
            pltpu.VMEM((L,), jnp.int32),
            pltpu.SemaphoreType.DMA,
        ],
    )(_sc_body)
    masks = k(view)
    bits = (masks[:, :, None] >> jnp.arange(6, dtype=jnp.int32)) & 1
    seen = jnp.max(bits, axis=(0, 1)).astype(jnp.float32)
    logits = jnp.zeros((1, 18), dtype=jnp.float32)
    return lax.dynamic_update_slice(logits, seen.reshape(1, 6), (0, 0))


def kernel(ram):
    return _run(ram)


# speedup vs baseline: 15.2395x; 1.0533x over previous
import functools

import jax
import jax.numpy as jnp
from jax import lax
from jax.experimental import pallas as pl
from jax.experimental.pallas import tpu as pltpu
from jax.experimental.pallas import tpu_sc as plsc

N_ROWS = 262144
N_COLS = 128
NUM_CORES = 2
NUM_SUBCORES = 16
NUM_WORKERS = NUM_CORES * NUM_SUBCORES
RPW = N_ROWS // NUM_WORKERS
L = 16
NCHUNK = 4
CH = RPW // NCHUNK
FIELDS = (32, 33, 34, 35, 0)


def _sc_body(view_hbm, out_hbm, *refs):
    (ia0, ib0, ic0, id0, ie0, ia1, ib1, ic1, id1, ie1,
     ba0, bb0, bc0, bd0, be0, ba1, bb1, bc1, bd1, be1,
     acc_v, sem0, sem1) = refs
    idx_sets = ((ia0, ib0, ic0, id0, ie0), (ia1, ib1, ic1, id1, ie1))
    buf_sets = ((ba0, bb0, bc0, bd0, be0), (ba1, bb1, bc1, bd1, be1))
    sems = (sem0, sem1)

    cid = lax.axis_index("c")
    sid = lax.axis_index("s")
    wid = sid * NUM_CORES + cid
    row0 = wid * RPW
    iot = lax.iota(jnp.int32, L)
    one = jnp.ones((L,), jnp.int32)

    def fill(c, idxs):
        def body(j, carry):
            base = N_COLS * (row0 + c * CH + j * L + iot)
            for f, idx in zip(FIELDS, idxs):
                idx[pl.ds(j * L, L)] = base + f
            return carry

        lax.fori_loop(0, CH // L, body, 0)

    def fire(idxs, bufs, sem):
        for idx, buf in zip(idxs, bufs):
            pltpu.async_copy(view_hbm.at[idx], buf, sem)

    def drain(idxs, bufs, sem):
        for idx, buf in zip(idxs, bufs):
            pltpu.make_async_copy(view_hbm.at[idx], buf, sem).wait()

    def compute(bufs, acc):
        bmx, bsx, bmy, bsy, bclk = bufs

        def body16(i, acc2):
            s = pl.ds(i * L, L)
            mi_x = bmx[s]
            su_x = bsx[s]
            mi_y = bmy[s]
            su_y = bsy[s]
            clk = bclk[s]
            dist_x = jnp.abs(su_x - mi_x)
            dist_y = jnp.abs(su_y - mi_y)
            go_down = su_y > mi_y + 1
            go_right = su_x > mi_x
            punch = (clk % 12) < 4
            d2 = dist_y <= 2
            act = jnp.where(go_down, 5, 2)
            act = jnp.where(d2 & (dist_x > 26), jnp.where(go_right, 3, 4), act)
            act = jnp.where(d2 & (dist_x < 23), jnp.where(go_right, 4, 3), act)
            act = jnp.where(
                d2 & (dist_x >= 23) & (dist_x <= 26), jnp.where(punch, 1, 0), act
            )
            return acc2 | (one << act)

        return lax.fori_loop(0, CH // L, body16, acc)

    fill(0, idx_sets[0])
    fire(idx_sets[0], buf_sets[0], sems[0])
    acc = jnp.zeros((L,), jnp.int32)
    for c in range(NCHUNK):
        if c + 1 < NCHUNK:
            nxt = (c + 1) % 2
            fill(c + 1, idx_sets[nxt])
            fire(idx_sets[nxt], buf_sets[nxt], sems[nxt])
        cur = c % 2
        drain(idx_sets[cur], buf_sets[cur], sems[cur])
        acc = compute(buf_sets[cur], acc)

    acc_v[...] = acc
    pltpu.sync_copy(acc_v, out_hbm.at[wid])


@jax.jit
def _run(ram):
    view = ram.reshape(N_ROWS * N_COLS)
    mesh = plsc.VectorSubcoreMesh(core_axis_name="c", subcore_axis_name="s")
    scratch = (
        [pltpu.VMEM((CH,), jnp.int32) for _ in range(20)]
        + [pltpu.VMEM((L,), jnp.int32)]
        + [pltpu.SemaphoreType.DMA, pltpu.SemaphoreType.DMA]
    )
    k = functools.partial(
        pl.kernel,
        mesh=mesh,
        out_type=jax.ShapeDtypeStruct((NUM_WORKERS, L), jnp.int32),
        scratch_types=scratch,
    )(_sc_body)
    masks = k(view)
    bits = (masks[:, :, None] >> jnp.arange(6, dtype=jnp.int32)) & 1
    seen = jnp.max(bits, axis=(0, 1)).astype(jnp.float32)
    logits = jnp.zeros((1, 18), dtype=jnp.float32)
    return lax.dynamic_update_slice(logits, seen.reshape(1, 6), (0, 0))


def kernel(ram):
    return _run(ram)


# speedup vs baseline: 17.8628x; 1.1721x over previous
import functools

import jax
import jax.numpy as jnp
from jax import lax
from jax.experimental import pallas as pl
from jax.experimental.pallas import tpu as pltpu
from jax.experimental.pallas import tpu_sc as plsc

N_ROWS = 262144
N_COLS = 128
NUM_CORES = 2
NUM_SUBCORES = 16
NUM_WORKERS = NUM_CORES * NUM_SUBCORES
RPW = N_ROWS // NUM_WORKERS
L = 16
NCHUNK = 8
CH = RPW // NCHUNK


def _dg(v, idx):
    return lax.gather(
        v,
        idx.reshape(L, 1),
        lax.GatherDimensionNumbers(
            offset_dims=(), collapsed_slice_dims=(0,), start_index_map=(0,)
        ),
        slice_sizes=(1,),
        mode=lax.GatherScatterMode.PROMISE_IN_BOUNDS,
    )


def _sc_body(view8_hbm, out_hbm, *refs):
    (ixy0, ixy1, iclk0, iclk1, bxy0, bxy1, bclk0, bclk1,
     acc_v, sem0, sem1) = refs
    idx_sets = ((ixy0, iclk0), (ixy1, iclk1))
    buf_sets = ((bxy0, bclk0), (bxy1, bclk1))
    sems = (sem0, sem1)

    cid = lax.axis_index("c")
    sid = lax.axis_index("s")
    wid = sid * NUM_CORES + cid
    row0 = wid * RPW
    iot = lax.iota(jnp.int32, L)
    one = jnp.ones((L,), jnp.int32)
    swap = iot ^ 1
    lane0 = jnp.zeros((L,), jnp.int32)
    lane2 = lane0 + 2

    def fill(c, idxs):
        ixy, iclk = idxs

        def body(j, carry):
            r8 = 8 * (row0 + c * CH + j * L + iot)
            ixy[pl.ds(j * L, L)] = r8 + 2
            iclk[pl.ds(j * L, L)] = r8
            return carry

        lax.fori_loop(0, CH // L, body, 0)

    def fire(idxs, bufs, sem):
        ixy, iclk = idxs
        bxy, bclk = bufs
        pltpu.async_copy(view8_hbm.at[ixy], bxy, sem)
        pltpu.async_copy(view8_hbm.at[iclk], bclk, sem)

    def drain(idxs, bufs, sem):
        ixy, iclk = idxs
        bxy, bclk = bufs
        pltpu.make_async_copy(view8_hbm.at[ixy], bxy, sem).wait()
        pltpu.make_async_copy(view8_hbm.at[iclk], bclk, sem).wait()

    def compute(bufs, acc):
        bxy, bclk = bufs

        def group(g, acc2):
            dx = lane0
            dy = lane0
            ck = lane0
            for j in range(L):
                vxy = bxy[g * L + j, :]
                vck = bclk[g * L + j, :]
                d = _dg(vxy, swap) - vxy
                m = iot == j
                dx = jnp.where(m, _dg(d, lane0), dx)
                dy = jnp.where(m, _dg(d, lane2), dy)
                ck = jnp.where(m, _dg(vck, lane0), ck)
            dist_x = jnp.abs(dx)
            dist_y = jnp.abs(dy)
            go_down = dy > 1
            go_right = dx > 0
            punch = (ck % 12) < 4
            d2 = dist_y <= 2
            act = jnp.where(go_down, 5, 2)
            act = jnp.where(d2 & (dist_x > 26), jnp.where(go_right, 3, 4), act)
            act = jnp.where(d2 & (dist_x < 23), jnp.where(go_right, 4, 3), act)
            act = jnp.where(
                d2 & (dist_x >= 23) & (dist_x <= 26), jnp.where(punch, 1, 0), act
            )
            return acc2 | (one << act)

        return lax.fori_loop(0, CH // L, group, acc)

    fill(0, idx_sets[0])
    fire(idx_sets[0], buf_sets[0], sems[0])
    acc = jnp.zeros((L,), jnp.int32)
    for c in range(NCHUNK):
        if c + 1 < NCHUNK:
            nxt = (c + 1) % 2
            fill(c + 1, idx_sets[nxt])
            fire(idx_sets[nxt], buf_sets[nxt], sems[nxt])
        cur = c % 2
        drain(idx_sets[cur], buf_sets[cur], sems[cur])
        acc = compute(buf_sets[cur], acc)

    acc_v[...] = acc
    pltpu.sync_copy(acc_v, out_hbm.at[wid])


@jax.jit
def _run(ram):
    view8 = ram.reshape(N_ROWS * 8, L)
    mesh = plsc.VectorSubcoreMesh(core_axis_name="c", subcore_axis_name="s")
    scratch = (
        [pltpu.VMEM((CH,), jnp.int32) for _ in range(4)]
        + [pltpu.VMEM((CH, L), jnp.int32) for _ in range(4)]
        + [pltpu.VMEM((L,), jnp.int32)]
        + [pltpu.SemaphoreType.DMA, pltpu.SemaphoreType.DMA]
    )
    k = functools.partial(
        pl.kernel,
        mesh=mesh,
        out_type=jax.ShapeDtypeStruct((NUM_WORKERS, L), jnp.int32),
        scratch_types=scratch,
        compiler_params=pltpu.CompilerParams(use_tc_tiling_on_sc=False),
    )(_sc_body)
    masks = k(view8)
    bits = (masks[:, :, None] >> jnp.arange(6, dtype=jnp.int32)) & 1
    seen = jnp.max(bits, axis=(0, 1)).astype(jnp.float32)
    logits = jnp.zeros((1, 18), dtype=jnp.float32)
    return lax.dynamic_update_slice(logits, seen.reshape(1, 6), (0, 0))


def kernel(ram):
    return _run(ram)


# speedup vs baseline: 21.4907x; 1.2031x over previous
import functools

import jax
import jax.numpy as jnp
from jax import lax
from jax.experimental import pallas as pl
from jax.experimental.pallas import tpu as pltpu
from jax.experimental.pallas import tpu_sc as plsc

N_ROWS = 262144
N_COLS = 128
NUM_CORES = 2
NUM_SUBCORES = 16
NUM_WORKERS = NUM_CORES * NUM_SUBCORES
RPW = N_ROWS // NUM_WORKERS
L = 16
NCHUNK = 8
CH = RPW // NCHUNK


def _dg(v, idx):
    return lax.gather(
        v,
        idx.reshape(L, 1),
        lax.GatherDimensionNumbers(
            offset_dims=(), collapsed_slice_dims=(0,), start_index_map=(0,)
        ),
        slice_sizes=(1,),
        mode=lax.GatherScatterMode.PROMISE_IN_BOUNDS,
    )


def _sc_body(view8_hbm, out_hbm, *refs):
    (ixy0, ixy1, iclk0, iclk1, bxy0, bxy1, bclk0, bclk1,
     acc_v, sem0, sem1) = refs
    idx_sets = ((ixy0, iclk0), (ixy1, iclk1))
    buf_sets = ((bxy0, bclk0), (bxy1, bclk1))
    sems = (sem0, sem1)

    cid = lax.axis_index("c")
    sid = lax.axis_index("s")
    wid = sid * NUM_CORES + cid
    row0 = wid * RPW
    iot = lax.iota(jnp.int32, L)
    one = jnp.ones((L,), jnp.int32)
    swap = iot ^ 1
    lane0 = jnp.zeros((L,), jnp.int32)
    lane2 = lane0 + 2

    def fill(c, idxs):
        ixy, iclk = idxs

        def body(j, carry):
            r8 = 8 * (row0 + c * CH + j * L + iot)
            ixy[pl.ds(j * L, L)] = r8 + 2
            iclk[pl.ds(j * L, L)] = r8
            return carry

        lax.fori_loop(0, CH // L, body, 0)

    def fire(idxs, bufs, sem):
        ixy, iclk = idxs
        bxy, bclk = bufs
        pltpu.async_copy(view8_hbm.at[ixy], bxy, sem)
        pltpu.async_copy(view8_hbm.at[iclk], bclk, sem)

    def drain(idxs, bufs, sem):
        ixy, iclk = idxs
        bxy, bclk = bufs
        pltpu.make_async_copy(view8_hbm.at[ixy], bxy, sem).wait()
        pltpu.make_async_copy(view8_hbm.at[iclk], bclk, sem).wait()

    qmask = tuple((iot >> 2) == q for q in range(4))
    rots = tuple((iot - 4 * r) & 15 for r in (1, 2, 3))
    pat0 = (iot & 3) * 4
    pat2 = pat0 + 2

    def compute(bufs, acc):
        bxy, bclk = bufs

        def group(g, acc2):
            dx = lane0
            dy = lane0
            ck = lane0
            for q in range(4):
                b = g * L + 4 * q
                m = bxy[b, :]
                mc = bclk[b, :]
                for r in (1, 2, 3):
                    m = jnp.where(qmask[r], _dg(bxy[b + r, :], rots[r - 1]), m)
                    mc = jnp.where(qmask[r], _dg(bclk[b + r, :], rots[r - 1]), mc)
                d = _dg(m, swap) - m
                dx = jnp.where(qmask[q], _dg(d, pat0), dx)
                dy = jnp.where(qmask[q], _dg(d, pat2), dy)
                ck = jnp.where(qmask[q], _dg(mc, pat0), ck)
            dist_x = jnp.abs(dx)
            dist_y = jnp.abs(dy)
            go_down = dy > 1
            go_right = dx > 0
            punch = (ck % 12) < 4
            d2 = dist_y <= 2
            act = jnp.where(go_down, 5, 2)
            act = jnp.where(d2 & (dist_x > 26), jnp.where(go_right, 3, 4), act)
            act = jnp.where(d2 & (dist_x < 23), jnp.where(go_right, 4, 3), act)
            act = jnp.where(
                d2 & (dist_x >= 23) & (dist_x <= 26), jnp.where(punch, 1, 0), act
            )
            return acc2 | (one << act)

        return lax.fori_loop(0, CH // L, group, acc)

    fill(0, idx_sets[0])
    fire(idx_sets[0], buf_sets[0], sems[0])
    acc = jnp.zeros((L,), jnp.int32)
    for c in range(NCHUNK):
        if c + 1 < NCHUNK:
            nxt = (c + 1) % 2
            fill(c + 1, idx_sets[nxt])
            fire(idx_sets[nxt], buf_sets[nxt], sems[nxt])
        cur = c % 2
        drain(idx_sets[cur], buf_sets[cur], sems[cur])
        acc = compute(buf_sets[cur], acc)

    acc_v[...] = acc
    pltpu.sync_copy(acc_v, out_hbm.at[wid])


@jax.jit
def _run(ram):
    view8 = ram.reshape(N_ROWS * 8, L)
    mesh = plsc.VectorSubcoreMesh(core_axis_name="c", subcore_axis_name="s")
    scratch = (
        [pltpu.VMEM((CH,), jnp.int32) for _ in range(4)]
        + [pltpu.VMEM((CH, L), jnp.int32) for _ in range(4)]
        + [pltpu.VMEM((L,), jnp.int32)]
        + [pltpu.SemaphoreType.DMA, pltpu.SemaphoreType.DMA]
    )
    k = functools.partial(
        pl.kernel,
        mesh=mesh,
        out_type=jax.ShapeDtypeStruct((NUM_WORKERS, L), jnp.int32),
        scratch_types=scratch,
        compiler_params=pltpu.CompilerParams(use_tc_tiling_on_sc=False),
    )(_sc_body)
    masks = k(view8)
    bits = (masks[:, :, None] >> jnp.arange(6, dtype=jnp.int32)) & 1
    seen = jnp.max(bits, axis=(0, 1)).astype(jnp.float32)
    logits = jnp.zeros((1, 18), dtype=jnp.float32)
    return lax.dynamic_update_slice(logits, seen.reshape(1, 6), (0, 0))


def kernel(ram):
    return _run(ram)
